# baseline (speedup 1.0000x reference)
import functools

import jax
import jax.numpy as jnp
from jax import lax
from jax.experimental import pallas as pl
from jax.experimental.pallas import tpu as pltpu

EPS = 1e-5
LANE = 128
TM = 512
TM_AFFINE = 2048
VMEM_LIMIT = 64 * 1024 * 1024


def _ru(v, m):
    return (v + m - 1) // m * m


def _mm_stats_body(p_ref, w_ref, y_ref, st_ref):
    y = jnp.dot(p_ref[...], w_ref[...], preferred_element_type=jnp.float32)
    y_ref[...] = y
    st_ref[...] = jnp.stack([jnp.sum(y, axis=0), jnp.sum(y * y, axis=0)])[None]


@functools.partial(jax.jit, static_argnames=("tm",))
def _conv_mm_stats(p, w, *, tm):
    mp, kp = p.shape
    ocp = w.shape[1]
    nt = mp // tm
    return pl.pallas_call(
        _mm_stats_body,
        out_shape=(jax.ShapeDtypeStruct((mp, ocp), jnp.float32),
                   jax.ShapeDtypeStruct((nt, 2, ocp), jnp.float32)),
        grid=(nt,),
        in_specs=[pl.BlockSpec((tm, kp), lambda i: (i, 0)),
                  pl.BlockSpec((kp, ocp), lambda i: (0, 0))],
        out_specs=(pl.BlockSpec((tm, ocp), lambda i: (i, 0)),
                   pl.BlockSpec((1, 2, ocp), lambda i: (i, 0, 0))),
        compiler_params=pltpu.CompilerParams(
            dimension_semantics=("parallel",),
            vmem_limit_bytes=VMEM_LIMIT),
    )(p, w)


def _affine_body(y_ref, a_ref, c_ref, o_ref):
    o_ref[...] = jnp.maximum(y_ref[...] * a_ref[...] + c_ref[...], 0.0)


@functools.partial(jax.jit, static_argnames=("tm",))
def _apply_affine(y, a, c, *, tm):
    mp, ocp = y.shape
    return pl.pallas_call(
        _affine_body,
        out_shape=jax.ShapeDtypeStruct((mp, ocp), jnp.float32),
        grid=(mp // tm,),
        in_specs=[pl.BlockSpec((tm, ocp), lambda i: (i, 0)),
                  pl.BlockSpec((1, ocp), lambda i: (0, 0)),
                  pl.BlockSpec((1, ocp), lambda i: (0, 0))],
        out_specs=pl.BlockSpec((tm, ocp), lambda i: (i, 0)),
        compiler_params=pltpu.CompilerParams(
            dimension_semantics=("parallel",),
            vmem_limit_bytes=VMEM_LIMIT),
    )(y, a, c)


def _seq_stats_body(p_ref, o_ref):
    nt = p_ref.shape[0]
    ocp = p_ref.shape[2]

    def body(i, acc):
        return acc + p_ref[i]

    o_ref[...] = lax.fori_loop(0, nt, body,
                               jnp.zeros((2, ocp), jnp.float32))


@jax.jit
def _run_seq_stats(parts):
    nt, _, ocp = parts.shape
    return pl.pallas_call(
        _seq_stats_body,
        out_shape=jax.ShapeDtypeStruct((2, ocp), jnp.float32),
        compiler_params=pltpu.CompilerParams(
            vmem_limit_bytes=VMEM_LIMIT),
    )(parts)


_TAPS = [(di, dj) for di in (-1, 0, 1) for dj in (-1, 0, 1)]


def _dconv3_compute(prev_ref, cur_ref, next_ref, w_ref, wpix, c, rb, himg):
    hwb = rb * wpix
    x = cur_ref[...].reshape(hwb, LANE)
    z1 = jnp.zeros((1, LANE), jnp.float32)
    k = himg // rb
    if k == 1:
        prow = jnp.zeros((wpix, LANE), jnp.float32)
        nrow = prow
    else:
        g = pl.program_id(0)
        prow = jnp.where((g % k) != 0, prev_ref[0], 0.0)
        nrow = jnp.where((g % k) != k - 1, next_ref[0], 0.0)
    xe = jnp.concatenate([z1, prow, x, nrow, z1], axis=0)
    jpos = lax.broadcasted_iota(jnp.int32, (hwb, 1), 0) & (wpix - 1)
    kp = w_ref.shape[0]
    op = None
    for t, (di, dj) in enumerate(_TAPS):
        a0 = 1 + wpix + di * wpix + dj
        s = xe[a0:a0 + hwb]
        if dj == 1:
            s = jnp.where(jpos != wpix - 1, s, 0.0)
        elif dj == -1:
            s = jnp.where(jpos != 0, s, 0.0)
        s = jnp.pad(s, ((0, 0), (0, kp - LANE)))
        if t:
            s = jnp.roll(s, t * c, axis=1)
        op = s if op is None else op + s
    return jnp.dot(op, w_ref[...], preferred_element_type=jnp.float32)


def _dconv3_body(prev_ref, cur_ref, next_ref, w_ref, y_ref, st_ref,
                 *, wpix, c, rb, himg):
    acc = _dconv3_compute(prev_ref, cur_ref, next_ref, w_ref,
                          wpix, c, rb, himg)
    y_ref[...] = acc
    hwb = rb * wpix
    tsub = TM if hwb % TM == 0 else hwb
    rows = []
    for i in range(hwb // tsub):
        t = acc[i * tsub:(i + 1) * tsub]
        rows.append(jnp.sum(t, axis=0))
    for i in range(hwb // tsub):
        t = acc[i * tsub:(i + 1) * tsub]
        rows.append(jnp.sum(t * t, axis=0))
    st_ref[...] = jnp.stack(rows)[None]


def _dconv3_bias_body(prev_ref, cur_ref, next_ref, w_ref, b_ref, y_ref,
                      *, wpix, c, rb, himg):
    acc = _dconv3_compute(prev_ref, cur_ref, next_ref, w_ref,
                          wpix, c, rb, himg)
    y_ref[...] = acc + b_ref[0, :]


def _halo_specs(wpix, rb, nh, kc, ocp):
    return [
        pl.BlockSpec((1, wpix, LANE),
                     lambda g: (jnp.maximum(g * rb - 1, 0), 0, 0)),
        pl.BlockSpec((rb, wpix, LANE), lambda g: (g, 0, 0)),
        pl.BlockSpec((1, wpix, LANE),
                     lambda g: (jnp.minimum(g * rb + rb, nh - 1), 0, 0)),
        pl.BlockSpec((kc, ocp), lambda g: (0, 0)),
    ]


def _pick_rb(wpix, himg):
    rb = max(1, 8192 // wpix)
    while himg % rb:
        rb //= 2
    return min(rb, himg)


@functools.partial(jax.jit, static_argnames=("wpix", "c", "himg"))
def _run_dconv3(xv, wmat, *, wpix, c, himg):
    nh = xv.shape[0]
    kc, ocp = wmat.shape
    rb = _pick_rb(wpix, himg)
    ng = nh // rb
    hwb = rb * wpix
    nsub = hwb // (TM if hwb % TM == 0 else hwb)
    return pl.pallas_call(
        functools.partial(_dconv3_body, wpix=wpix, c=c, rb=rb, himg=himg),
        out_shape=(jax.ShapeDtypeStruct((nh * wpix, ocp), jnp.float32),
                   jax.ShapeDtypeStruct((ng, 2 * nsub, ocp), jnp.float32)),
        grid=(ng,),
        in_specs=_halo_specs(wpix, rb, nh, kc, ocp),
        out_specs=(pl.BlockSpec((rb * wpix, ocp), lambda g: (g, 0)),
                   pl.BlockSpec((1, 2 * nsub, ocp), lambda g: (g, 0, 0))),
        compiler_params=pltpu.CompilerParams(
            dimension_semantics=("parallel",),
            vmem_limit_bytes=VMEM_LIMIT),
    )(xv, xv, xv, wmat)


@functools.partial(jax.jit, static_argnames=("wpix", "c", "himg"))
def _run_dconv3_bias(xv, wmat, b, *, wpix, c, himg):
    nh = xv.shape[0]
    kc, ocp = wmat.shape
    rb = _pick_rb(wpix, himg)
    ng = nh // rb
    specs = _halo_specs(wpix, rb, nh, kc, ocp)
    specs.append(pl.BlockSpec((1, ocp), lambda g: (0, 0)))
    return pl.pallas_call(
        functools.partial(_dconv3_bias_body, wpix=wpix, c=c, rb=rb,
                          himg=himg),
        out_shape=jax.ShapeDtypeStruct((nh * wpix, ocp), jnp.float32),
        grid=(ng,),
        in_specs=specs,
        out_specs=pl.BlockSpec((rb * wpix, ocp), lambda g: (g, 0)),
        compiler_params=pltpu.CompilerParams(
            dimension_semantics=("parallel",),
            vmem_limit_bytes=VMEM_LIMIT),
    )(xv, xv, xv, wmat, b)


def _im2col(x, kh, kw, sh, sw, ph, pw):
    n, h, w, c = x.shape
    if ph or pw:
        x = jnp.pad(x, ((0, 0), (ph, ph), (pw, pw), (0, 0)))
        h += 2 * ph
        w += 2 * pw
    oh = (h - kh) // sh + 1
    ow = (w - kw) // sw + 1
    taps = [x[:, i:i + sh * oh:sh, j:j + sw * ow:sw, :]
            for i in range(kh) for j in range(kw)]
    cols = taps[0] if len(taps) == 1 else jnp.concatenate(taps, axis=-1)
    return cols.reshape(n * oh * ow, kh * kw * c), (n, oh, ow)


def _bn_affine(y, parts, m, gamma, beta, oc, geom):
    ocp = y.shape[1]
    sq2 = _run_seq_stats(parts)
    ssum, ssq = sq2[0], sq2[1]
    mean = ssum / m
    var = jnp.maximum(ssq / m - mean * mean, 0.0)
    g = jnp.pad(gamma.astype(jnp.float32), (0, ocp - oc), constant_values=1.0)
    b = jnp.pad(beta.astype(jnp.float32), (0, ocp - oc))
    av = g * lax.rsqrt(var + EPS)
    cv = b - mean * av
    mp = y.shape[0]
    tm_a = next(t for t in (TM_AFFINE, TM, mp) if mp % t == 0)
    out = _apply_affine(y, av.reshape(1, ocp), cv.reshape(1, ocp), tm=tm_a)
    if mp != m:
        out = out[:m]
    return out, geom, oc


def _conv_bn_relu_4d(x, wt, gamma, beta, stride=(1, 1), padding=(0, 0)):
    oc, ic, kh, kw = wt.shape
    w2 = wt.transpose(2, 3, 1, 0).reshape(kh * kw * ic, oc).astype(jnp.float32)
    if kh == 1 and kw == 1 and stride == (1, 1) and padding == (0, 0):
        n, h, wd, c = x.shape
        pt, geom = x.reshape(n * h * wd, c), (n, h, wd)
    else:
        pt, geom = _im2col(x, kh, kw, stride[0], stride[1],
                           padding[0], padding[1])
    m, k = pt.shape
    kp = _ru(k, LANE)
    ocp = _ru(oc, LANE)
    mp = _ru(m, TM)
    p = jnp.pad(pt, ((0, mp - m), (0, kp - k)))
    wp = jnp.pad(w2, ((0, kp - k), (0, ocp - oc)))
    y, st = _conv_mm_stats(p, wp, tm=TM)
    return _bn_affine(y, st, m, gamma, beta, oc, geom)


def _sq_conv(t, wt, gamma, beta):
    flat, geom, c = t
    cp = flat.shape[1]
    oc = wt.shape[0]
    ocp = _ru(oc, LANE)
    w2 = jnp.zeros((cp, ocp), jnp.float32).at[:c, :oc].set(
        wt.reshape(oc, c).T.astype(jnp.float32))
    m = flat.shape[0]
    y, st = _conv_mm_stats(flat, w2, tm=TM if m % TM == 0 else m)
    return _bn_affine(y, st, m, gamma, beta, oc, geom)


def _e1e3_wmat(w1, w3):
    oc1, c = w1.shape[:2]
    oc3 = w3.shape[0]
    oc = oc1 + oc3
    kp = _ru(9 * c, LANE)
    ocp = _ru(oc, LANE)
    m3 = w3.transpose(2, 3, 1, 0).reshape(9 * c, oc3)
    wm = jnp.zeros((kp, ocp), jnp.float32)
    wm = wm.at[:9 * c, oc1:oc].set(m3)
    wm = wm.at[4 * c:5 * c, :oc1].set(w1.reshape(oc1, c).T)
    return wm


def _expand(prm, t):
    flat, (n, h, w), c = t
    w1, g1, b1 = prm["e1"]
    w3, g3, b3 = prm["e3"]
    oc = w1.shape[0] + w3.shape[0]
    wmat = _e1e3_wmat(w1.astype(jnp.float32), w3.astype(jnp.float32))
    y, st = _run_dconv3(flat.reshape(n * h, w, LANE), wmat,
                        wpix=w, c=c, himg=h)
    nsub = st.shape[1] // 2
    ocp = st.shape[2]
    parts = jnp.stack([st[:, :nsub, :].reshape(-1, ocp),
                       st[:, nsub:, :].reshape(-1, ocp)], axis=1)
    return _bn_affine(y, parts, n * h * w,
                      jnp.concatenate([g1, g3]), jnp.concatenate([b1, b3]),
                      oc, (n, h, w))


def _fire(prm, t):
    return _expand(prm, _sq_conv(t, *prm["sq"]))


def _deconv_bn_relu(t, wt, gamma, beta):
    flat, (n, h, wd), c = t
    x = flat[:, :c].reshape(n, h, wd, c)
    xz = jnp.zeros((n, h, 2 * wd - 1, c), x.dtype).at[:, :, ::2, :].set(x)
    xz = jnp.pad(xz, ((0, 0), (0, 0), (2, 2), (0, 0)))
    wf = jnp.flip(wt, axis=3).transpose(1, 0, 2, 3)
    return _conv_bn_relu_4d(xz, wf, gamma, beta)


def _fire_deconv(prm, t):
    s = _sq_conv(t, *prm["sq"])
    s = _deconv_bn_relu(s, *prm["de"])
    return _expand(prm, s)


def _maxpool(t):
    flat, (n, h, w), c = t
    cp = flat.shape[1]
    x = flat.reshape(n, h, w, cp)
    ow = -(-(w - 3) // 2) + 1
    if (ow - 1) * 2 >= w:
        ow -= 1
    pad_w = max((ow - 1) * 2 + 3 - w, 0)
    neg = jnp.asarray(-jnp.inf, x.dtype)
    xp = jnp.pad(x, ((0, 0), (1, 1), (0, pad_w), (0, 0)),
                 constant_values=neg)
    pooled = lax.reduce_window(xp, neg, lax.max,
                               (1, 3, 3, 1), (1, 1, 2, 1), "VALID")
    return pooled.reshape(n * h * ow, cp), (n, h, ow), c


def _add(t1, t2):
    f1, geom, c = t1
    f2 = t2[0]
    return f1 + f2, geom, c


def _c14_wmat(wt):
    oc, c = wt.shape[0], wt.shape[1]
    k = 9 * c
    wm = jnp.zeros((_ru(k, LANE), LANE), jnp.float32)
    return wm.at[:k, :oc].set(
        wt.transpose(2, 3, 1, 0).reshape(k, oc).astype(jnp.float32))


def _conv_bias(t, wt, bias):
    flat, (n, h, w), c = t
    wmat = _c14_wmat(wt)
    bvec = jnp.pad(bias.astype(jnp.float32), (0, LANE - 1)).reshape(1, LANE)
    y = _run_dconv3_bias(flat.reshape(n * h, w, LANE), wmat, bvec,
                         wpix=w, c=c, himg=h)
    return y[:, :1].reshape(n, h, w, 1)


_FIRES = ["fire2", "fire3", "fire4", "fire5",
          "fire6", "fire7", "fire8", "fire9"]
_DFIRES = ["fire10", "fire11", "fire12", "fire13"]


def kernel(x, *args):
    a = list(args)
    pos = 0

    def take():
        nonlocal pos
        t = (a[pos], a[pos + 1], a[pos + 2])
        pos += 3
        return t

    conv1 = take()
    skip_p = take()
    fp = {}
    for nm in _FIRES:
        fp[nm] = {"sq": take(), "e1": take(), "e3": take()}
    for nm in _DFIRES:
        fp[nm] = {"sq": take(), "de": take(), "e1": take(), "e3": take()}
    c14_w, c14_b = a[pos], a[pos + 1]

    out_c1 = _conv_bn_relu_4d(x, *conv1, stride=(1, 2), padding=(1, 1))
    skip = _conv_bn_relu_4d(x, *skip_p)
    out = _maxpool(out_c1)
    out_f3 = _fire(fp["fire3"], _fire(fp["fire2"], out))
    out = _maxpool(out_f3)
    out_f5 = _fire(fp["fire5"], _fire(fp["fire4"], out))
    out = _maxpool(out_f5)
    out = _fire(fp["fire9"],
                _fire(fp["fire8"],
                      _fire(fp["fire7"],
                            _fire(fp["fire6"], out))))
    out = _add(_fire_deconv(fp["fire10"], out), out_f5)
    out = _add(_fire_deconv(fp["fire11"], out), out_f3)
    out = _add(_fire_deconv(fp["fire12"], out), out_c1)
    out = _add(_fire_deconv(fp["fire13"], out), skip)
    return _conv_bias(out, c14_w, c14_b)


# speedup vs baseline: 1.0041x; 1.0041x over previous
import functools

import jax
import jax.numpy as jnp
from jax import lax
from jax.experimental import pallas as pl
from jax.experimental.pallas import tpu as pltpu

EPS = 1e-5
LANE = 128
TM = 512
TM_AFFINE = 2048
VMEM_LIMIT = 64 * 1024 * 1024


def _ru(v, m):
    return (v + m - 1) // m * m


def _mm_stats_body(p_ref, w_ref, y_ref, st_ref):
    y = jnp.dot(p_ref[...], w_ref[...], preferred_element_type=jnp.float32)
    y_ref[...] = y
    st_ref[...] = jnp.stack([jnp.sum(y, axis=0), jnp.sum(y * y, axis=0)])[None]


@functools.partial(jax.jit, static_argnames=("tm",))
def _conv_mm_stats(p, w, *, tm):
    mp, kp = p.shape
    ocp = w.shape[1]
    nt = mp // tm
    return pl.pallas_call(
        _mm_stats_body,
        out_shape=(jax.ShapeDtypeStruct((mp, ocp), jnp.float32),
                   jax.ShapeDtypeStruct((nt, 2, ocp), jnp.float32)),
        grid=(nt,),
        in_specs=[pl.BlockSpec((tm, kp), lambda i: (i, 0)),
                  pl.BlockSpec((kp, ocp), lambda i: (0, 0))],
        out_specs=(pl.BlockSpec((tm, ocp), lambda i: (i, 0)),
                   pl.BlockSpec((1, 2, ocp), lambda i: (i, 0, 0))),
        compiler_params=pltpu.CompilerParams(
            dimension_semantics=("parallel",),
            vmem_limit_bytes=VMEM_LIMIT),
    )(p, w)


def _affine_body(y_ref, a_ref, c_ref, o_ref):
    o_ref[...] = jnp.maximum(y_ref[...] * a_ref[...] + c_ref[...], 0.0)


@functools.partial(jax.jit, static_argnames=("tm",))
def _apply_affine(y, a, c, *, tm):
    mp, ocp = y.shape
    return pl.pallas_call(
        _affine_body,
        out_shape=jax.ShapeDtypeStruct((mp, ocp), jnp.float32),
        grid=(mp // tm,),
        in_specs=[pl.BlockSpec((tm, ocp), lambda i: (i, 0)),
                  pl.BlockSpec((1, ocp), lambda i: (0, 0)),
                  pl.BlockSpec((1, ocp), lambda i: (0, 0))],
        out_specs=pl.BlockSpec((tm, ocp), lambda i: (i, 0)),
        compiler_params=pltpu.CompilerParams(
            dimension_semantics=("parallel",),
            vmem_limit_bytes=VMEM_LIMIT),
    )(y, a, c)


def _seq_stats_body(p_ref, o_ref):
    nt = p_ref.shape[0]
    ocp = p_ref.shape[2]

    def body(i, acc):
        return acc + p_ref[i]

    o_ref[...] = lax.fori_loop(0, nt, body,
                               jnp.zeros((2, ocp), jnp.float32))


@jax.jit
def _run_seq_stats(parts):
    nt, _, ocp = parts.shape
    return pl.pallas_call(
        _seq_stats_body,
        out_shape=jax.ShapeDtypeStruct((2, ocp), jnp.float32),
        compiler_params=pltpu.CompilerParams(
            vmem_limit_bytes=VMEM_LIMIT),
    )(parts)


_TAPS = [(di, dj) for di in (-1, 0, 1) for dj in (-1, 0, 1)]


def _dconv3_compute(prev_ref, cur_ref, next_ref, w_ref, wpix, c, rb, himg):
    hwb = rb * wpix
    x = cur_ref[...].reshape(hwb, LANE)
    z1 = jnp.zeros((1, LANE), jnp.float32)
    k = himg // rb
    if k == 1:
        prow = jnp.zeros((wpix, LANE), jnp.float32)
        nrow = prow
    else:
        g = pl.program_id(0)
        prow = jnp.where((g % k) != 0, prev_ref[0], 0.0)
        nrow = jnp.where((g % k) != k - 1, next_ref[0], 0.0)
    xe = jnp.concatenate([z1, prow, x, nrow, z1], axis=0)
    jpos = lax.broadcasted_iota(jnp.int32, (hwb, 1), 0) & (wpix - 1)
    kp = w_ref.shape[0]
    op = None
    for t, (di, dj) in enumerate(_TAPS):
        a0 = 1 + wpix + di * wpix + dj
        s = xe[a0:a0 + hwb]
        if dj == 1:
            s = jnp.where(jpos != wpix - 1, s, 0.0)
        elif dj == -1:
            s = jnp.where(jpos != 0, s, 0.0)
        s = jnp.pad(s, ((0, 0), (0, kp - LANE)))
        if t:
            s = jnp.roll(s, t * c, axis=1)
        op = s if op is None else op + s
    return jnp.dot(op, w_ref[...], preferred_element_type=jnp.float32)


def _dconv3_body(prev_ref, cur_ref, next_ref, w_ref, y_ref, st_ref,
                 *, wpix, c, rb, himg):
    acc = _dconv3_compute(prev_ref, cur_ref, next_ref, w_ref,
                          wpix, c, rb, himg)
    y_ref[...] = acc
    hwb = rb * wpix
    tsub = TM if hwb % TM == 0 else hwb
    rows = []
    for i in range(hwb // tsub):
        t = acc[i * tsub:(i + 1) * tsub]
        rows.append(jnp.sum(t, axis=0))
    for i in range(hwb // tsub):
        t = acc[i * tsub:(i + 1) * tsub]
        rows.append(jnp.sum(t * t, axis=0))
    st_ref[...] = jnp.stack(rows)[None]


def _dconv3_bias_body(prev_ref, cur_ref, next_ref, w_ref, b_ref, y_ref,
                      *, wpix, c, rb, himg):
    acc = _dconv3_compute(prev_ref, cur_ref, next_ref, w_ref,
                          wpix, c, rb, himg)
    y_ref[...] = acc + b_ref[0, :]


def _halo_specs(wpix, rb, nh, kc, ocp):
    return [
        pl.BlockSpec((1, wpix, LANE),
                     lambda g: (jnp.maximum(g * rb - 1, 0), 0, 0)),
        pl.BlockSpec((rb, wpix, LANE), lambda g: (g, 0, 0)),
        pl.BlockSpec((1, wpix, LANE),
                     lambda g: (jnp.minimum(g * rb + rb, nh - 1), 0, 0)),
        pl.BlockSpec((kc, ocp), lambda g: (0, 0)),
    ]


def _pick_rb(wpix, himg):
    rb = max(1, 4096 // wpix)
    while himg % rb:
        rb //= 2
    return min(rb, himg)


@functools.partial(jax.jit, static_argnames=("wpix", "c", "himg"))
def _run_dconv3(xv, wmat, *, wpix, c, himg):
    nh = xv.shape[0]
    kc, ocp = wmat.shape
    rb = _pick_rb(wpix, himg)
    ng = nh // rb
    hwb = rb * wpix
    nsub = hwb // (TM if hwb % TM == 0 else hwb)
    return pl.pallas_call(
        functools.partial(_dconv3_body, wpix=wpix, c=c, rb=rb, himg=himg),
        out_shape=(jax.ShapeDtypeStruct((nh * wpix, ocp), jnp.float32),
                   jax.ShapeDtypeStruct((ng, 2 * nsub, ocp), jnp.float32)),
        grid=(ng,),
        in_specs=_halo_specs(wpix, rb, nh, kc, ocp),
        out_specs=(pl.BlockSpec((rb * wpix, ocp), lambda g: (g, 0)),
                   pl.BlockSpec((1, 2 * nsub, ocp), lambda g: (g, 0, 0))),
        compiler_params=pltpu.CompilerParams(
            dimension_semantics=("parallel",),
            vmem_limit_bytes=VMEM_LIMIT),
    )(xv, xv, xv, wmat)


@functools.partial(jax.jit, static_argnames=("wpix", "c", "himg"))
def _run_dconv3_bias(xv, wmat, b, *, wpix, c, himg):
    nh = xv.shape[0]
    kc, ocp = wmat.shape
    rb = _pick_rb(wpix, himg)
    ng = nh // rb
    specs = _halo_specs(wpix, rb, nh, kc, ocp)
    specs.append(pl.BlockSpec((1, ocp), lambda g: (0, 0)))
    return pl.pallas_call(
        functools.partial(_dconv3_bias_body, wpix=wpix, c=c, rb=rb,
                          himg=himg),
        out_shape=jax.ShapeDtypeStruct((nh * wpix, ocp), jnp.float32),
        grid=(ng,),
        in_specs=specs,
        out_specs=pl.BlockSpec((rb * wpix, ocp), lambda g: (g, 0)),
        compiler_params=pltpu.CompilerParams(
            dimension_semantics=("parallel",),
            vmem_limit_bytes=VMEM_LIMIT),
    )(xv, xv, xv, wmat, b)


def _im2col(x, kh, kw, sh, sw, ph, pw):
    n, h, w, c = x.shape
    if ph or pw:
        x = jnp.pad(x, ((0, 0), (ph, ph), (pw, pw), (0, 0)))
        h += 2 * ph
        w += 2 * pw
    oh = (h - kh) // sh + 1
    ow = (w - kw) // sw + 1
    taps = [x[:, i:i + sh * oh:sh, j:j + sw * ow:sw, :]
            for i in range(kh) for j in range(kw)]
    cols = taps[0] if len(taps) == 1 else jnp.concatenate(taps, axis=-1)
    return cols.reshape(n * oh * ow, kh * kw * c), (n, oh, ow)


def _bn_affine(y, parts, m, gamma, beta, oc, geom):
    ocp = y.shape[1]
    sq2 = _run_seq_stats(parts)
    ssum, ssq = sq2[0], sq2[1]
    mean = ssum / m
    var = jnp.maximum(ssq / m - mean * mean, 0.0)
    g = jnp.pad(gamma.astype(jnp.float32), (0, ocp - oc), constant_values=1.0)
    b = jnp.pad(beta.astype(jnp.float32), (0, ocp - oc))
    av = g * lax.rsqrt(var + EPS)
    cv = b - mean * av
    mp = y.shape[0]
    tm_a = next(t for t in (TM_AFFINE, TM, mp) if mp % t == 0)
    out = _apply_affine(y, av.reshape(1, ocp), cv.reshape(1, ocp), tm=tm_a)
    if mp != m:
        out = out[:m]
    return out, geom, oc


def _conv_bn_relu_4d(x, wt, gamma, beta, stride=(1, 1), padding=(0, 0)):
    oc, ic, kh, kw = wt.shape
    w2 = wt.transpose(2, 3, 1, 0).reshape(kh * kw * ic, oc).astype(jnp.float32)
    if kh == 1 and kw == 1 and stride == (1, 1) and padding == (0, 0):
        n, h, wd, c = x.shape
        pt, geom = x.reshape(n * h * wd, c), (n, h, wd)
    else:
        pt, geom = _im2col(x, kh, kw, stride[0], stride[1],
                           padding[0], padding[1])
    m, k = pt.shape
    kp = _ru(k, LANE)
    ocp = _ru(oc, LANE)
    mp = _ru(m, TM)
    p = jnp.pad(pt, ((0, mp - m), (0, kp - k)))
    wp = jnp.pad(w2, ((0, kp - k), (0, ocp - oc)))
    y, st = _conv_mm_stats(p, wp, tm=TM)
    return _bn_affine(y, st, m, gamma, beta, oc, geom)


def _sq_conv(t, wt, gamma, beta):
    flat, geom, c = t
    cp = flat.shape[1]
    oc = wt.shape[0]
    ocp = _ru(oc, LANE)
    w2 = jnp.zeros((cp, ocp), jnp.float32).at[:c, :oc].set(
        wt.reshape(oc, c).T.astype(jnp.float32))
    m = flat.shape[0]
    y, st = _conv_mm_stats(flat, w2, tm=TM if m % TM == 0 else m)
    return _bn_affine(y, st, m, gamma, beta, oc, geom)


def _e1e3_wmat(w1, w3):
    oc1, c = w1.shape[:2]
    oc3 = w3.shape[0]
    oc = oc1 + oc3
    kp = _ru(9 * c, LANE)
    ocp = _ru(oc, LANE)
    m3 = w3.transpose(2, 3, 1, 0).reshape(9 * c, oc3)
    wm = jnp.zeros((kp, ocp), jnp.float32)
    wm = wm.at[:9 * c, oc1:oc].set(m3)
    wm = wm.at[4 * c:5 * c, :oc1].set(w1.reshape(oc1, c).T)
    return wm


def _expand(prm, t):
    flat, (n, h, w), c = t
    w1, g1, b1 = prm["e1"]
    w3, g3, b3 = prm["e3"]
    oc = w1.shape[0] + w3.shape[0]
    wmat = _e1e3_wmat(w1.astype(jnp.float32), w3.astype(jnp.float32))
    y, st = _run_dconv3(flat.reshape(n * h, w, LANE), wmat,
                        wpix=w, c=c, himg=h)
    nsub = st.shape[1] // 2
    ocp = st.shape[2]
    parts = jnp.stack([st[:, :nsub, :].reshape(-1, ocp),
                       st[:, nsub:, :].reshape(-1, ocp)], axis=1)
    return _bn_affine(y, parts, n * h * w,
                      jnp.concatenate([g1, g3]), jnp.concatenate([b1, b3]),
                      oc, (n, h, w))


def _fire(prm, t):
    return _expand(prm, _sq_conv(t, *prm["sq"]))


def _deconv_bn_relu(t, wt, gamma, beta):
    flat, (n, h, wd), c = t
    x = flat[:, :c].reshape(n, h, wd, c)
    xz = jnp.zeros((n, h, 2 * wd - 1, c), x.dtype).at[:, :, ::2, :].set(x)
    xz = jnp.pad(xz, ((0, 0), (0, 0), (2, 2), (0, 0)))
    wf = jnp.flip(wt, axis=3).transpose(1, 0, 2, 3)
    return _conv_bn_relu_4d(xz, wf, gamma, beta)


def _fire_deconv(prm, t):
    s = _sq_conv(t, *prm["sq"])
    s = _deconv_bn_relu(s, *prm["de"])
    return _expand(prm, s)


def _maxpool(t):
    flat, (n, h, w), c = t
    cp = flat.shape[1]
    x = flat.reshape(n, h, w, cp)
    ow = -(-(w - 3) // 2) + 1
    if (ow - 1) * 2 >= w:
        ow -= 1
    pad_w = max((ow - 1) * 2 + 3 - w, 0)
    neg = jnp.asarray(-jnp.inf, x.dtype)
    xp = jnp.pad(x, ((0, 0), (1, 1), (0, pad_w), (0, 0)),
                 constant_values=neg)
    pooled = lax.reduce_window(xp, neg, lax.max,
                               (1, 3, 3, 1), (1, 1, 2, 1), "VALID")
    return pooled.reshape(n * h * ow, cp), (n, h, ow), c


def _add(t1, t2):
    f1, geom, c = t1
    f2 = t2[0]
    return f1 + f2, geom, c


def _c14_wmat(wt):
    oc, c = wt.shape[0], wt.shape[1]
    k = 9 * c
    wm = jnp.zeros((_ru(k, LANE), LANE), jnp.float32)
    return wm.at[:k, :oc].set(
        wt.transpose(2, 3, 1, 0).reshape(k, oc).astype(jnp.float32))


def _conv_bias(t, wt, bias):
    flat, (n, h, w), c = t
    wmat = _c14_wmat(wt)
    bvec = jnp.pad(bias.astype(jnp.float32), (0, LANE - 1)).reshape(1, LANE)
    y = _run_dconv3_bias(flat.reshape(n * h, w, LANE), wmat, bvec,
                         wpix=w, c=c, himg=h)
    return y[:, :1].reshape(n, h, w, 1)


_FIRES = ["fire2", "fire3", "fire4", "fire5",
          "fire6", "fire7", "fire8", "fire9"]
_DFIRES = ["fire10", "fire11", "fire12", "fire13"]


def kernel(x, *args):
    a = list(args)
    pos = 0

    def take():
        nonlocal pos
        t = (a[pos], a[pos + 1], a[pos + 2])
        pos += 3
        return t

    conv1 = take()
    skip_p = take()
    fp = {}
    for nm in _FIRES:
        fp[nm] = {"sq": take(), "e1": take(), "e3": take()}
    for nm in _DFIRES:
        fp[nm] = {"sq": take(), "de": take(), "e1": take(), "e3": take()}
    c14_w, c14_b = a[pos], a[pos + 1]

    out_c1 = _conv_bn_relu_4d(x, *conv1, stride=(1, 2), padding=(1, 1))
    skip = _conv_bn_relu_4d(x, *skip_p)
    out = _maxpool(out_c1)
    out_f3 = _fire(fp["fire3"], _fire(fp["fire2"], out))
    out = _maxpool(out_f3)
    out_f5 = _fire(fp["fire5"], _fire(fp["fire4"], out))
    out = _maxpool(out_f5)
    out = _fire(fp["fire9"],
                _fire(fp["fire8"],
                      _fire(fp["fire7"],
                            _fire(fp["fire6"], out))))
    out = _add(_fire_deconv(fp["fire10"], out), out_f5)
    out = _add(_fire_deconv(fp["fire11"], out), out_f3)
    out = _add(_fire_deconv(fp["fire12"], out), out_c1)
    out = _add(_fire_deconv(fp["fire13"], out), skip)
    return _conv_bias(out, c14_w, c14_b)
